# initial kernel scaffold (unmeasured)
import jax
import jax.numpy as jnp
from jax import lax
from jax.experimental import pallas as pl
from jax.experimental.pallas import tpu as pltpu

N_DEV = 8
E_LOCAL = 2
N_EXP = N_DEV * E_LOCAL


def kernel(x, router, W1, W2):
    t_loc, d = x.shape
    f = W1.shape[2]

    def body(x_ref, router_ref, w1_ref, w2_ref, out_ref,
             x_send, x_buf, r_buf, w_send, w_buf, p_send, p_buf,
             x_ssem, x_rsem, r_ssem, r_rsem,
             w_ssem, w_rsem, p_ssem, p_rsem):
        my = lax.axis_index("i")

        barrier_sem = pltpu.get_barrier_semaphore()
        for k in range(1, N_DEV):
            peer = lax.rem(my + k, N_DEV)
            pl.semaphore_signal(
                barrier_sem, inc=1,
                device_id=(peer,), device_id_type=pl.DeviceIdType.MESH,
            )
        pl.semaphore_wait(barrier_sem, N_DEV - 1)

        x_send[...] = x_ref[...].astype(jnp.bfloat16)
        x_sends, r_sends = [], []
        for k in range(1, N_DEV):
            peer = lax.rem(my + k, N_DEV)
            dx = pltpu.make_async_remote_copy(
                src_ref=x_send, dst_ref=x_buf.at[my],
                send_sem=x_ssem.at[k], recv_sem=x_rsem.at[my],
                device_id=(peer,), device_id_type=pl.DeviceIdType.MESH,
            )
            dx.start()
            x_sends.append(dx)
            dr = pltpu.make_async_remote_copy(
                src_ref=router_ref, dst_ref=r_buf.at[my],
                send_sem=r_ssem.at[k], recv_sem=r_rsem.at[my],
                device_id=(peer,), device_id_type=pl.DeviceIdType.MESH,
            )
            dr.start()
            r_sends.append(dr)

        for k in range(1, N_DEV):
            s = lax.rem(my + k, N_DEV)
            pltpu.make_async_remote_copy(
                src_ref=router_ref, dst_ref=r_buf.at[s],
                send_sem=r_ssem.at[k], recv_sem=r_rsem.at[s],
                device_id=(s,), device_id_type=pl.DeviceIdType.MESH,
            ).wait_recv()

        r_all = lax.dynamic_update_slice(
            r_buf[...], router_ref[...][None], (my, 0, 0)
        )
        gates = jnp.concatenate(
            [
                jnp.dot(x_ref[...], r_all[k],
                        precision=lax.Precision.HIGHEST,
                        preferred_element_type=jnp.float32)
                for k in range(N_DEV)
            ],
            axis=1,
        )

        eidx = lax.broadcasted_iota(jnp.int32, (t_loc, N_EXP), 1)
        m1 = jnp.max(gates, axis=1, keepdims=True)
        i1 = jnp.min(jnp.where(gates == m1, eidx, N_EXP), axis=1, keepdims=True)
        g2 = jnp.where(eidx == i1, jnp.float32(-1e30), gates)
        m2 = jnp.max(g2, axis=1, keepdims=True)
        i2 = jnp.min(jnp.where(g2 == m2, eidx, N_EXP), axis=1, keepdims=True)
        e2 = jnp.exp(m2 - m1)
        wt1 = 1.0 / (1.0 + e2)
        wt2 = e2 / (1.0 + e2)
        w_loc = (jnp.where(eidx == i1, wt1, 0.0)
                 + jnp.where(eidx == i2, wt2, 0.0))

        w_send[...] = w_loc
        w_sends = []
        for k in range(1, N_DEV):
            peer = lax.rem(my + k, N_DEV)
            dw = pltpu.make_async_remote_copy(
                src_ref=w_send, dst_ref=w_buf.at[my],
                send_sem=w_ssem.at[k], recv_sem=w_rsem.at[my],
                device_id=(peer,), device_id_type=pl.DeviceIdType.MESH,
            )
            dw.start()
            w_sends.append(dw)

        for k in range(1, N_DEV):
            s = lax.rem(my + k, N_DEV)
            pltpu.make_async_remote_copy(
                src_ref=x_send, dst_ref=x_buf.at[s],
                send_sem=x_ssem.at[k], recv_sem=x_rsem.at[s],
                device_id=(s,), device_id_type=pl.DeviceIdType.MESH,
            ).wait_recv()
            pltpu.make_async_remote_copy(
                src_ref=w_send, dst_ref=w_buf.at[s],
                send_sem=w_ssem.at[k], recv_sem=w_rsem.at[s],
                device_id=(s,), device_id_type=pl.DeviceIdType.MESH,
            ).wait_recv()

        x_all = lax.dynamic_update_slice(
            x_buf[...], x_send[...][None], (my, 0, 0)
        )
        xg = x_all.reshape(N_DEV * t_loc, d)
        w_all = lax.dynamic_update_slice(
            w_buf[...], w_loc[None], (my, 0, 0)
        )
        w_full = w_all.reshape(N_DEV * t_loc, N_EXP)

        acc = jnp.zeros((N_DEV * t_loc, d), jnp.float32)
        for e in range(E_LOCAL):
            h = jnp.dot(xg, w1_ref[e].astype(jnp.bfloat16),
                        preferred_element_type=jnp.float32)
            h = jnp.maximum(h, 0.0).astype(jnp.bfloat16)
            y = jnp.dot(h, w2_ref[e].astype(jnp.bfloat16),
                        preferred_element_type=jnp.float32)
            wcol = lax.dynamic_slice(
                w_full, (0, my * E_LOCAL + e), (N_DEV * t_loc, 1)
            )
            acc = acc + y * wcol
        p_send[...] = acc.reshape(N_DEV, t_loc, d).astype(jnp.bfloat16)

        p_sends = []
        for k in range(1, N_DEV):
            peer = lax.rem(my + k, N_DEV)
            dp = pltpu.make_async_remote_copy(
                src_ref=p_send.at[peer], dst_ref=p_buf.at[my],
                send_sem=p_ssem.at[k], recv_sem=p_rsem.at[my],
                device_id=(peer,), device_id_type=pl.DeviceIdType.MESH,
            )
            dp.start()
            p_sends.append(dp)
        for k in range(1, N_DEV):
            s = lax.rem(my + k, N_DEV)
            pltpu.make_async_remote_copy(
                src_ref=p_send.at[s], dst_ref=p_buf.at[s],
                send_sem=p_ssem.at[k], recv_sem=p_rsem.at[s],
                device_id=(s,), device_id_type=pl.DeviceIdType.MESH,
            ).wait_recv()

        own = lax.dynamic_slice(p_send[...], (my, 0, 0), (1, t_loc, d))
        p_all = lax.dynamic_update_slice(p_buf[...], own, (my, 0, 0))
        out_ref[...] = jnp.sum(p_all.astype(jnp.float32), axis=0)

        for dsc in x_sends + r_sends + w_sends + p_sends:
            dsc.wait_send()

    return pl.pallas_call(
        body,
        out_shape=jax.ShapeDtypeStruct((t_loc, d), jnp.float32),
        in_specs=[pl.BlockSpec(memory_space=pltpu.VMEM)] * 4,
        out_specs=pl.BlockSpec(memory_space=pltpu.VMEM),
        scratch_shapes=[
            pltpu.VMEM((t_loc, d), jnp.bfloat16),
            pltpu.VMEM((N_DEV, t_loc, d), jnp.bfloat16),
            pltpu.VMEM((N_DEV, d, E_LOCAL), jnp.float32),
            pltpu.VMEM((t_loc, N_EXP), jnp.float32),
            pltpu.VMEM((N_DEV, t_loc, N_EXP), jnp.float32),
            pltpu.VMEM((N_DEV, t_loc, d), jnp.bfloat16),
            pltpu.VMEM((N_DEV, t_loc, d), jnp.bfloat16),
        ] + [pltpu.SemaphoreType.DMA((N_DEV,))] * 8,
        compiler_params=pltpu.CompilerParams(collective_id=0),
    )(x, router, W1, W2)


# baseline (device time: 45343 ns/iter reference)
import jax
import jax.numpy as jnp
from jax import lax
from jax.experimental import pallas as pl
from jax.experimental.pallas import tpu as pltpu

N_DEV = 8
E_LOCAL = 2
N_EXP = N_DEV * E_LOCAL


def kernel(x, router, W1, W2):
    t_loc, d = x.shape
    f = W1.shape[2]

    def body(x_ref, router_ref, w1_ref, w2_ref, out_ref,
             x_send, x_buf, r_buf, w_send, w_buf, p_send, p_buf,
             x_ssem, x_rsem, r_ssem, r_rsem,
             w_ssem, w_rsem, p_ssem, p_rsem):
        my = lax.axis_index("i")

        barrier_sem = pltpu.get_barrier_semaphore()
        for k in range(1, N_DEV):
            peer = lax.rem(my + k, N_DEV)
            pl.semaphore_signal(
                barrier_sem, inc=1,
                device_id=(peer,), device_id_type=pl.DeviceIdType.MESH,
            )
        pl.semaphore_wait(barrier_sem, N_DEV - 1)

        x_send[...] = x_ref[...].astype(jnp.bfloat16)
        x_sends, r_sends = [], []
        for k in range(1, N_DEV):
            peer = lax.rem(my + k, N_DEV)
            dx = pltpu.make_async_remote_copy(
                src_ref=x_send, dst_ref=x_buf.at[my],
                send_sem=x_ssem.at[k], recv_sem=x_rsem.at[my],
                device_id=(peer,), device_id_type=pl.DeviceIdType.MESH,
            )
            dx.start()
            x_sends.append(dx)
            dr = pltpu.make_async_remote_copy(
                src_ref=router_ref, dst_ref=r_buf.at[my],
                send_sem=r_ssem.at[k], recv_sem=r_rsem.at[my],
                device_id=(peer,), device_id_type=pl.DeviceIdType.MESH,
            )
            dr.start()
            r_sends.append(dr)

        for k in range(1, N_DEV):
            s = lax.rem(my + k, N_DEV)
            pltpu.make_async_remote_copy(
                src_ref=router_ref, dst_ref=r_buf.at[s],
                send_sem=r_ssem.at[k], recv_sem=r_rsem.at[s],
                device_id=(s,), device_id_type=pl.DeviceIdType.MESH,
            ).wait_recv()

        blk = lax.broadcasted_iota(jnp.int32, (N_DEV, 1, 1), 0)
        r_all = jnp.where(blk == my, router_ref[...][None], r_buf[...])
        gates = jnp.concatenate(
            [
                jnp.dot(x_ref[...], r_all[k],
                        precision=lax.Precision.HIGHEST,
                        preferred_element_type=jnp.float32)
                for k in range(N_DEV)
            ],
            axis=1,
        )

        eidx = lax.broadcasted_iota(jnp.int32, (t_loc, N_EXP), 1)
        m1 = jnp.max(gates, axis=1, keepdims=True)
        i1 = jnp.min(jnp.where(gates == m1, eidx, N_EXP), axis=1, keepdims=True)
        g2 = jnp.where(eidx == i1, jnp.float32(-1e30), gates)
        m2 = jnp.max(g2, axis=1, keepdims=True)
        i2 = jnp.min(jnp.where(g2 == m2, eidx, N_EXP), axis=1, keepdims=True)
        e2 = jnp.exp(m2 - m1)
        wt1 = 1.0 / (1.0 + e2)
        wt2 = e2 / (1.0 + e2)
        w_loc = (jnp.where(eidx == i1, wt1, 0.0)
                 + jnp.where(eidx == i2, wt2, 0.0))

        w_send[...] = w_loc
        w_sends = []
        for k in range(1, N_DEV):
            peer = lax.rem(my + k, N_DEV)
            dw = pltpu.make_async_remote_copy(
                src_ref=w_send, dst_ref=w_buf.at[my],
                send_sem=w_ssem.at[k], recv_sem=w_rsem.at[my],
                device_id=(peer,), device_id_type=pl.DeviceIdType.MESH,
            )
            dw.start()
            w_sends.append(dw)

        for k in range(1, N_DEV):
            s = lax.rem(my + k, N_DEV)
            pltpu.make_async_remote_copy(
                src_ref=x_send, dst_ref=x_buf.at[s],
                send_sem=x_ssem.at[k], recv_sem=x_rsem.at[s],
                device_id=(s,), device_id_type=pl.DeviceIdType.MESH,
            ).wait_recv()
            pltpu.make_async_remote_copy(
                src_ref=w_send, dst_ref=w_buf.at[s],
                send_sem=w_ssem.at[k], recv_sem=w_rsem.at[s],
                device_id=(s,), device_id_type=pl.DeviceIdType.MESH,
            ).wait_recv()

        x_all = jnp.where(blk == my, x_send[...][None], x_buf[...])
        xg = x_all.reshape(N_DEV * t_loc, d)
        w_all = jnp.where(blk == my, w_loc[None], w_buf[...])
        w_full = w_all.reshape(N_DEV * t_loc, N_EXP)

        acc = jnp.zeros((N_DEV * t_loc, d), jnp.float32)
        for e in range(E_LOCAL):
            h = jnp.dot(xg, w1_ref[e].astype(jnp.bfloat16),
                        preferred_element_type=jnp.float32)
            h = jnp.maximum(h, 0.0).astype(jnp.bfloat16)
            y = jnp.dot(h, w2_ref[e].astype(jnp.bfloat16),
                        preferred_element_type=jnp.float32)
            col_iota = lax.broadcasted_iota(
                jnp.int32, (N_DEV * t_loc, N_EXP), 1
            )
            wcol = jnp.sum(
                jnp.where(col_iota == my * E_LOCAL + e, w_full, 0.0),
                axis=1, keepdims=True,
            )
            acc = acc + y * wcol
        p_send[...] = acc.reshape(N_DEV, t_loc, d).astype(jnp.bfloat16)

        p_sends = []
        for k in range(1, N_DEV):
            peer = lax.rem(my + k, N_DEV)
            dp = pltpu.make_async_remote_copy(
                src_ref=p_send.at[peer], dst_ref=p_buf.at[my],
                send_sem=p_ssem.at[k], recv_sem=p_rsem.at[my],
                device_id=(peer,), device_id_type=pl.DeviceIdType.MESH,
            )
            dp.start()
            p_sends.append(dp)
        for k in range(1, N_DEV):
            s = lax.rem(my + k, N_DEV)
            pltpu.make_async_remote_copy(
                src_ref=p_send.at[s], dst_ref=p_buf.at[s],
                send_sem=p_ssem.at[k], recv_sem=p_rsem.at[s],
                device_id=(s,), device_id_type=pl.DeviceIdType.MESH,
            ).wait_recv()

        p_all = jnp.where(blk == my, p_send[...], p_buf[...])
        out_ref[...] = jnp.sum(p_all.astype(jnp.float32), axis=0)

        for dsc in x_sends + r_sends + w_sends + p_sends:
            dsc.wait_send()

    return pl.pallas_call(
        body,
        out_shape=jax.ShapeDtypeStruct((t_loc, d), jnp.float32),
        in_specs=[pl.BlockSpec(memory_space=pltpu.VMEM)] * 4,
        out_specs=pl.BlockSpec(memory_space=pltpu.VMEM),
        scratch_shapes=[
            pltpu.VMEM((t_loc, d), jnp.bfloat16),
            pltpu.VMEM((N_DEV, t_loc, d), jnp.bfloat16),
            pltpu.VMEM((N_DEV, d, E_LOCAL), jnp.float32),
            pltpu.VMEM((t_loc, N_EXP), jnp.float32),
            pltpu.VMEM((N_DEV, t_loc, N_EXP), jnp.float32),
            pltpu.VMEM((N_DEV, t_loc, d), jnp.bfloat16),
            pltpu.VMEM((N_DEV, t_loc, d), jnp.bfloat16),
        ] + [pltpu.SemaphoreType.DMA((N_DEV,))] * 8,
        compiler_params=pltpu.CompilerParams(collective_id=0),
    )(x, router, W1, W2)


# device time: 25651 ns/iter; 1.7677x vs baseline; 1.7677x over previous
import jax
import jax.numpy as jnp
from jax import lax
from jax.experimental import pallas as pl
from jax.experimental.pallas import tpu as pltpu

N_DEV = 8
E_LOCAL = 2
N_EXP = N_DEV * E_LOCAL
CAP = 32


def kernel(x, router, W1, W2):
    t_loc, d = x.shape

    def body(x_ref, router_ref, w1_ref, w2_ref, out_ref,
             r_buf, xd_send, xd_buf, wd_send, wd_buf, p_send, p_buf,
             r_ssem, r_rsem, x_ssem, x_rsem,
             w_ssem, w_rsem, p_ssem, p_rsem):
        my = lax.axis_index("i")

        barrier_sem = pltpu.get_barrier_semaphore()
        for j in range(1, N_DEV):
            peer = lax.rem(my + j, N_DEV)
            pl.semaphore_signal(
                barrier_sem, inc=1,
                device_id=(peer,), device_id_type=pl.DeviceIdType.MESH,
            )
        pl.semaphore_wait(barrier_sem, N_DEV - 1)

        r_buf[0] = jnp.transpose(router_ref[...], (1, 0))
        sends = []
        for j in range(1, N_DEV):
            peer = lax.rem(my + j, N_DEV)
            dr = pltpu.make_async_remote_copy(
                src_ref=r_buf.at[0], dst_ref=r_buf.at[N_DEV - j],
                send_sem=r_ssem.at[j], recv_sem=r_rsem.at[N_DEV - j],
                device_id=(peer,), device_id_type=pl.DeviceIdType.MESH,
            )
            dr.start()
            sends.append(dr)
        for j in range(1, N_DEV):
            pltpu.make_async_remote_copy(
                src_ref=r_buf.at[0], dst_ref=r_buf.at[j],
                send_sem=r_ssem.at[j], recv_sem=r_rsem.at[j],
                device_id=(my,), device_id_type=pl.DeviceIdType.MESH,
            ).wait_recv()

        rt = r_buf[...].reshape(N_EXP, d)
        gates = lax.dot_general(
            x_ref[...], rt, (((1,), (1,)), ((), ())),
            precision=lax.Precision.HIGHEST,
            preferred_element_type=jnp.float32,
        )

        eidx = lax.broadcasted_iota(jnp.int32, (t_loc, N_EXP), 1)
        m1 = jnp.max(gates, axis=1, keepdims=True)
        i1 = jnp.min(jnp.where(gates == m1, eidx, N_EXP), axis=1, keepdims=True)
        g2 = jnp.where(eidx == i1, jnp.float32(-1e30), gates)
        m2 = jnp.max(g2, axis=1, keepdims=True)
        i2 = jnp.min(jnp.where(g2 == m2, eidx, N_EXP), axis=1, keepdims=True)
        e2 = jnp.exp(m2 - m1)
        wt1 = 1.0 / (1.0 + e2)
        wt2 = e2 / (1.0 + e2)
        w_loc = (jnp.where(eidx == i1, wt1, 0.0)
                 + jnp.where(eidx == i2, wt2, 0.0))

        o1 = i1 // E_LOCAL
        o2 = i2 // E_LOCAL
        jidx = lax.broadcasted_iota(jnp.int32, (t_loc, N_DEV), 1)
        msk = jnp.logical_or(jidx == o1, jidx == o2)
        tr = lax.broadcasted_iota(jnp.int32, (t_loc, t_loc), 0)
        tc = lax.broadcasted_iota(jnp.int32, (t_loc, t_loc), 1)
        lt = jnp.where(tc < tr, 1.0, 0.0).astype(jnp.float32)
        rank = lax.dot_general(
            lt, msk.astype(jnp.float32), (((1,), (0,)), ((), ())),
            preferred_element_type=jnp.float32,
        )

        cidx = lax.broadcasted_iota(jnp.int32, (t_loc, CAP), 1).astype(jnp.float32)
        x_b16 = x_ref[...].astype(jnp.bfloat16)
        dts = []
        for j in range(N_DEV):
            dt = jnp.where(
                jnp.logical_and(msk[:, j:j + 1], rank[:, j:j + 1] == cidx),
                1.0, 0.0,
            ).astype(jnp.bfloat16)
            dts.append(dt)
        dcat = jnp.concatenate(dts, axis=1)
        xd_all = lax.dot_general(
            dcat, x_b16, (((0,), (0,)), ((), ())),
            preferred_element_type=jnp.float32,
        ).astype(jnp.bfloat16)
        wd_all = lax.dot_general(
            dcat, w_loc.astype(jnp.bfloat16), (((0,), (0,)), ((), ())),
            preferred_element_type=jnp.float32,
        )
        for j in range(N_DEV):
            xd_j = xd_all[j * CAP:(j + 1) * CAP]
            wd_j = wd_all[j * CAP:(j + 1) * CAP]
            if j == 0:
                xd_buf[0] = xd_j
                wd_buf[0] = wd_j
            else:
                xd_send[j] = xd_j
                wd_send[j] = wd_j

        for j in range(1, N_DEV):
            peer = lax.rem(my + j, N_DEV)
            dx = pltpu.make_async_remote_copy(
                src_ref=xd_send.at[j], dst_ref=xd_buf.at[N_DEV - j],
                send_sem=x_ssem.at[j], recv_sem=x_rsem.at[N_DEV - j],
                device_id=(peer,), device_id_type=pl.DeviceIdType.MESH,
            )
            dx.start()
            sends.append(dx)
            dw = pltpu.make_async_remote_copy(
                src_ref=wd_send.at[j], dst_ref=wd_buf.at[N_DEV - j],
                send_sem=w_ssem.at[j], recv_sem=w_rsem.at[N_DEV - j],
                device_id=(peer,), device_id_type=pl.DeviceIdType.MESH,
            )
            dw.start()
            sends.append(dw)
        for j in range(1, N_DEV):
            pltpu.make_async_remote_copy(
                src_ref=xd_send.at[j], dst_ref=xd_buf.at[j],
                send_sem=x_ssem.at[j], recv_sem=x_rsem.at[j],
                device_id=(my,), device_id_type=pl.DeviceIdType.MESH,
            ).wait_recv()
            pltpu.make_async_remote_copy(
                src_ref=wd_send.at[j], dst_ref=wd_buf.at[j],
                send_sem=w_ssem.at[j], recv_sem=w_rsem.at[j],
                device_id=(my,), device_id_type=pl.DeviceIdType.MESH,
            ).wait_recv()

        t_cap = N_DEV * CAP
        xg = xd_buf[...].reshape(t_cap, d)
        wg = jnp.concatenate(
            [
                wd_buf[i][:, E_LOCAL * ((N_DEV - i) % N_DEV):
                          E_LOCAL * ((N_DEV - i) % N_DEV) + E_LOCAL]
                for i in range(N_DEV)
            ],
            axis=0,
        )
        acc = jnp.zeros((t_cap, d), jnp.float32)
        for e in range(E_LOCAL):
            h = jnp.dot(xg, w1_ref[e].astype(jnp.bfloat16),
                        preferred_element_type=jnp.float32)
            h = jnp.maximum(h, 0.0).astype(jnp.bfloat16)
            y = jnp.dot(h, w2_ref[e].astype(jnp.bfloat16),
                        preferred_element_type=jnp.float32)
            acc = acc + y * wg[:, e:e + 1]
        p_send[...] = acc.reshape(N_DEV, CAP, d).astype(jnp.bfloat16)
        p_buf[0] = p_send[0]

        for j in range(1, N_DEV):
            peer = lax.rem(my + j, N_DEV)
            dp = pltpu.make_async_remote_copy(
                src_ref=p_send.at[j], dst_ref=p_buf.at[N_DEV - j],
                send_sem=p_ssem.at[j], recv_sem=p_rsem.at[N_DEV - j],
                device_id=(peer,), device_id_type=pl.DeviceIdType.MESH,
            )
            dp.start()
            sends.append(dp)
        for j in range(1, N_DEV):
            pltpu.make_async_remote_copy(
                src_ref=p_send.at[j], dst_ref=p_buf.at[j],
                send_sem=p_ssem.at[j], recv_sem=p_rsem.at[j],
                device_id=(my,), device_id_type=pl.DeviceIdType.MESH,
            ).wait_recv()

        out_ref[...] = lax.dot_general(
            dcat, p_buf[...].reshape(t_cap, d), (((1,), (0,)), ((), ())),
            preferred_element_type=jnp.float32,
        )

        for dsc in sends:
            dsc.wait_send()

    return pl.pallas_call(
        body,
        out_shape=jax.ShapeDtypeStruct((t_loc, d), jnp.float32),
        in_specs=[pl.BlockSpec(memory_space=pltpu.VMEM)] * 4,
        out_specs=pl.BlockSpec(memory_space=pltpu.VMEM),
        scratch_shapes=[
            pltpu.VMEM((N_DEV, E_LOCAL, d), jnp.float32),
            pltpu.VMEM((N_DEV, CAP, d), jnp.bfloat16),
            pltpu.VMEM((N_DEV, CAP, d), jnp.bfloat16),
            pltpu.VMEM((N_DEV, CAP, N_EXP), jnp.float32),
            pltpu.VMEM((N_DEV, CAP, N_EXP), jnp.float32),
            pltpu.VMEM((N_DEV, CAP, d), jnp.bfloat16),
            pltpu.VMEM((N_DEV, CAP, d), jnp.bfloat16),
        ] + [pltpu.SemaphoreType.DMA((N_DEV,))] * 8,
        compiler_params=pltpu.CompilerParams(collective_id=0),
    )(x, router, W1, W2)
